# baseline (speedup 1.0000x reference)
import jax
import jax.numpy as jnp
from jax.experimental import pallas as pl
from jax.experimental.pallas import tpu as pltpu

N = 5000
S = 16
H = 64
GD = 4 * H
B = 2560
GRID = 2
NP = GRID * B


def _lstm_block_kernel(x_ref, w0_ref, b0_ref, w1_ref, b1_ref, hw_ref,
                       out_ref):
    w0 = w0_ref[...]
    b0 = b0_ref[...]
    w1 = w1_ref[...]
    b1 = b1_ref[...]

    def cell(w, b, xin, h, c):
        z = jnp.concatenate([xin, h], axis=0).astype(jnp.bfloat16)
        gates = b + jnp.dot(w, z, preferred_element_type=jnp.float32)
        i = jax.nn.sigmoid(gates[0:H])
        f = jax.nn.sigmoid(gates[H:2 * H])
        g = jnp.tanh(gates[2 * H:3 * H])
        o = jax.nn.sigmoid(gates[3 * H:4 * H])
        c = f * c + i * g
        h = o * jnp.tanh(c)
        return h, c

    z = jnp.zeros((H, B), jnp.float32)
    h0, c0, h1, c1 = z, z, z, z
    for t in range(S):
        xt = x_ref[:, t, :].T
        h0, c0 = cell(w0, b0, xt, h0, c0)
        h1, c1 = cell(w1, b1, h0, h1, c1)

    out_ref[...] = jnp.dot(hw_ref[...], h1,
                           preferred_element_type=jnp.float32)


def kernel(data, label, proposals, classes,
           w_ih_0, w_hh_0, b_ih_0, b_hh_0,
           w_ih_1, w_hh_1, b_ih_1, b_hh_1,
           cls_w, cls_b, bbox_w, bbox_b):
    f32 = jnp.float32
    w0 = jnp.concatenate([w_ih_0, w_hh_0], axis=1).astype(jnp.bfloat16)
    w1 = jnp.concatenate([w_ih_1, w_hh_1], axis=1).astype(jnp.bfloat16)
    b0 = jnp.tile((b_ih_0 + b_hh_0).reshape(GD, 1), (1, B))
    b1 = jnp.tile((b_ih_1 + b_hh_1).reshape(GD, 1), (1, B))
    hw = jnp.concatenate([cls_w, bbox_w, jnp.zeros((1, H), f32)], axis=0)

    out = pl.pallas_call(
        _lstm_block_kernel,
        grid=(GRID,),
        in_specs=[
            pl.BlockSpec((B, S, H), lambda i: (i, 0, 0)),
            pl.BlockSpec((GD, 2 * H), lambda i: (0, 0)),
            pl.BlockSpec((GD, B), lambda i: (0, 0)),
            pl.BlockSpec((GD, 2 * H), lambda i: (0, 0)),
            pl.BlockSpec((GD, B), lambda i: (0, 0)),
            pl.BlockSpec((8, H), lambda i: (0, 0)),
        ],
        out_specs=pl.BlockSpec((8, B), lambda i: (0, i)),
        out_shape=jax.ShapeDtypeStruct((8, NP), f32),
        compiler_params=pltpu.CompilerParams(
            dimension_semantics=("parallel",)),
    )(proposals, w0, b0, w1, b1, hw)

    outT = out.T[:N]
    cls_feat = outT[:, :5] + cls_b
    bbox_feat = outT[:, 5:7] + bbox_b
    return (cls_feat, bbox_feat, jnp.float32(0.0), jnp.float32(0.0))


# speedup vs baseline: 1.0558x; 1.0558x over previous
import jax
import jax.numpy as jnp
from jax.experimental import pallas as pl
from jax.experimental.pallas import tpu as pltpu

N = 5000
S = 16
H = 64
GD = 4 * H
B = 2560
GRID = 2
OVER = GRID * B - N


def _lstm_block_kernel(x_hbm, w0_ref, b0_ref, w1_ref, b1_ref, hw_ref,
                       out_ref, xbuf, sem):
    i = pl.program_id(0)
    base = jnp.where(i == 0, 0, N - B)

    def copy(t):
        return pltpu.make_async_copy(
            x_hbm.at[pl.ds(base, B), pl.ds(t, 1)],
            xbuf.at[t % 2],
            sem.at[t % 2])

    copy(0).start()

    w0 = w0_ref[...]
    b0 = b0_ref[...]
    w1 = w1_ref[...]
    b1 = b1_ref[...]

    def cell(w, b, xin, h, c):
        z = jnp.concatenate([xin, h], axis=0).astype(jnp.bfloat16)
        gates = b + jnp.dot(w, z, preferred_element_type=jnp.float32)
        i_ = jax.nn.sigmoid(gates[0:H])
        f_ = jax.nn.sigmoid(gates[H:2 * H])
        g_ = jnp.tanh(gates[2 * H:3 * H])
        o_ = jax.nn.sigmoid(gates[3 * H:4 * H])
        c = f_ * c + i_ * g_
        h = o_ * jnp.tanh(c)
        return h, c

    z = jnp.zeros((H, B), jnp.float32)
    h0, c0, h1, c1 = z, z, z, z
    for t in range(S):
        if t + 1 < S:
            copy(t + 1).start()
        copy(t).wait()
        xt = xbuf[t % 2].reshape(B, H).T
        h0, c0 = cell(w0, b0, xt, h0, c0)
        h1, c1 = cell(w1, b1, h0, h1, c1)

    out_ref[...] = jnp.dot(hw_ref[...], h1,
                           preferred_element_type=jnp.float32)


def kernel(data, label, proposals, classes,
           w_ih_0, w_hh_0, b_ih_0, b_hh_0,
           w_ih_1, w_hh_1, b_ih_1, b_hh_1,
           cls_w, cls_b, bbox_w, bbox_b):
    f32 = jnp.float32
    w0 = jnp.concatenate([w_ih_0, w_hh_0], axis=1).astype(jnp.bfloat16)
    w1 = jnp.concatenate([w_ih_1, w_hh_1], axis=1).astype(jnp.bfloat16)
    b0 = jnp.tile((b_ih_0 + b_hh_0).reshape(GD, 1), (1, B))
    b1 = jnp.tile((b_ih_1 + b_hh_1).reshape(GD, 1), (1, B))
    hw = jnp.concatenate([cls_w, bbox_w, jnp.zeros((1, H), f32)], axis=0)

    out = pl.pallas_call(
        _lstm_block_kernel,
        grid=(GRID,),
        in_specs=[
            pl.BlockSpec(memory_space=pltpu.MemorySpace.HBM),
            pl.BlockSpec((GD, 2 * H), lambda i: (0, 0)),
            pl.BlockSpec((GD, B), lambda i: (0, 0)),
            pl.BlockSpec((GD, 2 * H), lambda i: (0, 0)),
            pl.BlockSpec((GD, B), lambda i: (0, 0)),
            pl.BlockSpec((8, H), lambda i: (0, 0)),
        ],
        out_specs=pl.BlockSpec((8, B), lambda i: (0, i)),
        out_shape=jax.ShapeDtypeStruct((8, GRID * B), f32),
        scratch_shapes=[
            pltpu.VMEM((2, B, 1, 1, H), f32),
            pltpu.SemaphoreType.DMA((2,)),
        ],
        compiler_params=pltpu.CompilerParams(
            dimension_semantics=("parallel",)),
    )(proposals.reshape(N, S, 1, H), w0, b0, w1, b1, hw)

    outT = jnp.concatenate([out[:, :B], out[:, B + OVER:]], axis=1).T
    cls_feat = outT[:, :5] + cls_b
    bbox_feat = outT[:, 5:7] + bbox_b
    return (cls_feat, bbox_feat, jnp.float32(0.0), jnp.float32(0.0))


# speedup vs baseline: 1.2907x; 1.2224x over previous
import jax
import jax.numpy as jnp
from jax.experimental import pallas as pl
from jax.experimental.pallas import tpu as pltpu

N = 5000
S = 16
H = 64
GD = 4 * H
B = 2560
GRID = 2
OVER = GRID * B - N


def _lstm_block_kernel(x_hbm, w0_ref, b0_ref, w1_ref, b1_ref, hw_ref,
                       out_ref, xbuf, sem):
    blk = pl.program_id(0)
    base = jnp.where(blk == 0, 0, N - B)

    def copy(t):
        return pltpu.make_async_copy(
            x_hbm.at[pl.ds(base, B), t],
            xbuf.at[t % 2],
            sem.at[t % 2])

    copy(0).start()

    w0 = w0_ref[...]
    b0 = b0_ref[...]
    w1 = w1_ref[...]
    b1 = b1_ref[...]

    def cell(w, b, xt, h, c):
        z = jnp.concatenate([xt, h], axis=1).astype(jnp.bfloat16)
        gates = b + jnp.dot(z, w, preferred_element_type=jnp.float32)
        sig_if = jax.nn.sigmoid(gates[:, 0:2 * H])
        t_go = jnp.tanh(gates[:, 2 * H:4 * H])
        i_ = sig_if[:, 0:H]
        f_ = sig_if[:, H:2 * H]
        g_ = t_go[:, 0:H]
        o_ = t_go[:, H:2 * H] * 0.5 + 0.5
        c = f_ * c + i_ * g_
        h = o_ * jnp.tanh(c)
        return h, c

    z = jnp.zeros((B, H), jnp.float32)
    h0, c0, h1, c1 = z, z, z, z
    for t in range(S):
        if t + 1 < S:
            copy(t + 1).start()
        copy(t).wait()
        xt = xbuf[t % 2]
        h0, c0 = cell(w0, b0, xt, h0, c0)
        h1, c1 = cell(w1, b1, h0, h1, c1)

    out_ref[...] = jnp.dot(h1, hw_ref[...],
                           preferred_element_type=jnp.float32)


def kernel(data, label, proposals, classes,
           w_ih_0, w_hh_0, b_ih_0, b_hh_0,
           w_ih_1, w_hh_1, b_ih_1, b_hh_1,
           cls_w, cls_b, bbox_w, bbox_b):
    f32 = jnp.float32

    def prep(w_ih, w_hh, b_ih, b_hh):
        wt = jnp.concatenate([w_ih, w_hh], axis=1).T
        scale = jnp.concatenate([jnp.ones((3 * H,), f32),
                                 jnp.full((H,), 0.5, f32)])
        wt = wt * scale[None, :]
        b = ((b_ih + b_hh) * scale).reshape(1, GD)
        return wt.astype(jnp.bfloat16), b

    w0, b0 = prep(w_ih_0, w_hh_0, b_ih_0, b_hh_0)
    w1, b1 = prep(w_ih_1, w_hh_1, b_ih_1, b_hh_1)
    hw = jnp.concatenate([cls_w, bbox_w, jnp.zeros((1, H), f32)], axis=0).T

    out = pl.pallas_call(
        _lstm_block_kernel,
        grid=(GRID,),
        in_specs=[
            pl.BlockSpec(memory_space=pltpu.MemorySpace.HBM),
            pl.BlockSpec((2 * H, GD), lambda i: (0, 0)),
            pl.BlockSpec((1, GD), lambda i: (0, 0)),
            pl.BlockSpec((2 * H, GD), lambda i: (0, 0)),
            pl.BlockSpec((1, GD), lambda i: (0, 0)),
            pl.BlockSpec((H, 8), lambda i: (0, 0)),
        ],
        out_specs=pl.BlockSpec((B, 8), lambda i: (i, 0)),
        out_shape=jax.ShapeDtypeStruct((GRID * B, 8), f32),
        scratch_shapes=[
            pltpu.VMEM((2, B, H), f32),
            pltpu.SemaphoreType.DMA((2,)),
        ],
        compiler_params=pltpu.CompilerParams(
            dimension_semantics=("parallel",)),
    )(proposals, w0, b0, w1, b1, hw)

    outN = jnp.concatenate([out[:B], out[B + OVER:]], axis=0)
    cls_feat = outN[:, :5] + cls_b
    bbox_feat = outN[:, 5:7] + bbox_b
    return (cls_feat, bbox_feat, jnp.float32(0.0), jnp.float32(0.0))
